# baseline (speedup 1.0000x reference)
import jax
import jax.numpy as jnp
from jax import lax
from jax.experimental import pallas as pl
from jax.experimental.pallas import tpu as pltpu
from jax.experimental.pallas import tpu_sc as plsc

NC = 2
NS = 16
NW = NC * NS
L = 16

V = 100000
F = 100000
E = 3200000

F_PAD = 100352
SLICE = F_PAD // NS

EPW = E // NW
CH = 2000
NCHUNK = EPW // CH


def _sc_body(gm_hbm, ef_hbm, vp_hbm, out_hbm,
             vp_v, vidx0, vidx1, fidx0, fidx1, ef0, ef1, sat0, sat1,
             zbuf, shared, sem_in0, sem_in1, sem_sc0, sem_sc1):
  c = lax.axis_index("c")
  s = lax.axis_index("s")
  wid = c * NS + s
  base0 = wid * EPW
  vidx_v = (vidx0, vidx1)
  fidx_v = (fidx0, fidx1)
  ef_v = (ef0, ef1)
  sat_v = (sat0, sat1)
  sem_in = (sem_in0, sem_in1)
  sem_sc = (sem_sc0, sem_sc1)

  def start_in(ci, slot):
    b = base0 + ci * CH
    pltpu.async_copy(gm_hbm.at[pl.ds(b, CH)], vidx_v[slot], sem_in[slot])
    pltpu.async_copy(gm_hbm.at[pl.ds(E + b, CH)], fidx_v[slot], sem_in[slot])
    pltpu.async_copy(ef_hbm.at[pl.ds(b, CH)], ef_v[slot], sem_in[slot])

  def wait_in(slot):
    pltpu.make_async_copy(gm_hbm.at[pl.ds(0, CH)], vidx_v[slot],
                          sem_in[slot]).wait()
    pltpu.make_async_copy(gm_hbm.at[pl.ds(0, CH)], fidx_v[slot],
                          sem_in[slot]).wait()
    pltpu.make_async_copy(ef_hbm.at[pl.ds(0, CH)], ef_v[slot],
                          sem_in[slot]).wait()

  def compute(slot):
    @plsc.parallel_loop(0, CH // L, unroll=2)
    def _lanes(j):
      o = j * L
      vals = plsc.load_gather(vp_v, [vidx_v[slot][pl.ds(o, L)]])
      ef = ef_v[slot][pl.ds(o, L)]
      ev = ef * vals + (1.0 - ef) * 0.5
      sat_v[slot][pl.ds(o, L)] = jnp.where(ev > 0.5, 1.0, 0.0).astype(
          jnp.float32)

  def start_scatter(slot):
    pltpu.async_copy(sat_v[slot], shared.at[fidx_v[slot]], sem_sc[slot],
                     add=True)

  def wait_scatter(slot):
    pltpu.make_async_copy(sat_v[slot], shared.at[fidx_v[slot]],
                          sem_sc[slot]).wait()

  @pl.loop(0, SLICE // L)
  def _zero(i):
    zbuf[pl.ds(i * L, L)] = jnp.zeros((L,), jnp.float32)

  pltpu.sync_copy(zbuf, shared.at[pl.ds(s * SLICE, SLICE)])

  start_in(0, 0)
  pltpu.sync_copy(vp_hbm, vp_v)
  plsc.subcore_barrier()

  wait_in(0)
  compute(0)
  start_in(1, 1)
  start_scatter(0)

  @pl.loop(1, NCHUNK - 1, step=2)
  def _pair(ci0):
    for b in range(2):
      ci = ci0 + b
      cur = (1 + b) % 2
      oth = 1 - cur
      wait_in(cur)
      compute(cur)
      wait_scatter(oth)
      start_in(ci + 1, oth)
      start_scatter(cur)

  wait_in(1)
  compute(1)
  wait_scatter(0)
  start_scatter(1)
  wait_scatter(1)

  plsc.subcore_barrier()
  pltpu.sync_copy(shared.at[pl.ds(s * SLICE, SLICE)], zbuf)
  pltpu.sync_copy(zbuf, out_hbm.at[pl.ds(c * F_PAD + s * SLICE, SLICE)])


def _tc_body(p_ref, bfm_ref, is_sat_ref, unsat_ref):
  nrow = F_PAD // 128
  total = p_ref[:nrow, :] + p_ref[nrow:, :]
  in_batch = bfm_ref[...] == 0
  clause_sat = jnp.logical_and(total > 0.0, in_batch)
  batch_values = jnp.sum(clause_sat.astype(jnp.int32))
  max_sat = jnp.sum(in_batch.astype(jnp.int32))
  is_sat_ref[...] = jnp.where(batch_values == max_sat, 1.0, 0.0).reshape(1, 1)
  unsat_ref[...] = (max_sat - batch_values).astype(jnp.float32).reshape(1, 1)


@jax.jit
def kernel(variable_prediction, label, graph_map, batch_variable_map,
           batch_function_map, edge_feature, meta_data):
  del label, batch_variable_map, meta_data
  vp = variable_prediction.reshape(V)
  ef = edge_feature.reshape(E)
  gm = graph_map.reshape(2 * E)

  assert NCHUNK % 2 == 0

  mesh = plsc.VectorSubcoreMesh(core_axis_name="c", subcore_axis_name="s")
  partials = pl.kernel(
      _sc_body,
      out_type=jax.ShapeDtypeStruct((NC * F_PAD,), jnp.float32),
      mesh=mesh,
      compiler_params=pltpu.CompilerParams(needs_layout_passes=False),
      scratch_types=[
          pltpu.VMEM((V,), jnp.float32),
          pltpu.VMEM((CH,), jnp.int32),
          pltpu.VMEM((CH,), jnp.int32),
          pltpu.VMEM((CH,), jnp.int32),
          pltpu.VMEM((CH,), jnp.int32),
          pltpu.VMEM((CH,), jnp.float32),
          pltpu.VMEM((CH,), jnp.float32),
          pltpu.VMEM((CH,), jnp.float32),
          pltpu.VMEM((CH,), jnp.float32),
          pltpu.VMEM((SLICE,), jnp.float32),
          pltpu.VMEM_SHARED((F_PAD,), jnp.float32),
          pltpu.SemaphoreType.DMA,
          pltpu.SemaphoreType.DMA,
          pltpu.SemaphoreType.DMA,
          pltpu.SemaphoreType.DMA,
      ],
  )(gm, ef, vp)

  bfm = jnp.pad(batch_function_map, (0, F_PAD - F), constant_values=1)

  is_sat, unsat = pl.pallas_call(
      _tc_body,
      out_shape=(
          jax.ShapeDtypeStruct((1, 1), jnp.float32),
          jax.ShapeDtypeStruct((1, 1), jnp.float32),
      ),
  )(partials.reshape(2 * F_PAD // 128, 128),
    bfm.reshape(F_PAD // 128, 128))
  return is_sat, unsat


# speedup vs baseline: 1.0707x; 1.0707x over previous
import jax
import jax.numpy as jnp
from jax import lax
from jax.experimental import pallas as pl
from jax.experimental.pallas import tpu as pltpu
from jax.experimental.pallas import tpu_sc as plsc

NC = 2
NS = 16
NW = NC * NS
L = 16

V = 100000
F = 100000
E = 3200000

F_PAD = 100352
SLICE = F_PAD // NS

EPW = E // NW
CH = 2000
NCHUNK = EPW // CH


def _sc_body(gm_hbm, ef_hbm, vp_hbm, out_hbm,
             vp_v, vidx0, vidx1, fidx0, fidx1, ef0, ef1, sat0, sat1,
             zbuf, shared, sem_in0, sem_in1, sem_sc0, sem_sc1, sem_vp):
  c = lax.axis_index("c")
  s = lax.axis_index("s")
  wid = c * NS + s
  base0 = wid * EPW
  vidx_v = (vidx0, vidx1)
  fidx_v = (fidx0, fidx1)
  ef_v = (ef0, ef1)
  sat_v = (sat0, sat1)
  sem_in = (sem_in0, sem_in1)
  sem_sc = (sem_sc0, sem_sc1)

  def start_in(ci, slot):
    b = base0 + ci * CH
    pltpu.async_copy(gm_hbm.at[pl.ds(b, CH)], vidx_v[slot], sem_in[slot])
    pltpu.async_copy(gm_hbm.at[pl.ds(E + b, CH)], fidx_v[slot], sem_in[slot])
    pltpu.async_copy(ef_hbm.at[pl.ds(b, CH)], ef_v[slot], sem_in[slot])

  def wait_in(slot):
    pltpu.make_async_copy(gm_hbm.at[pl.ds(0, CH)], vidx_v[slot],
                          sem_in[slot]).wait()
    pltpu.make_async_copy(gm_hbm.at[pl.ds(0, CH)], fidx_v[slot],
                          sem_in[slot]).wait()
    pltpu.make_async_copy(ef_hbm.at[pl.ds(0, CH)], ef_v[slot],
                          sem_in[slot]).wait()

  def compute(slot):
    @plsc.parallel_loop(0, CH // L, unroll=4)
    def _lanes(j):
      o = j * L
      vals = plsc.load_gather(vp_v, [vidx_v[slot][pl.ds(o, L)]])
      ef = ef_v[slot][pl.ds(o, L)]
      ev = ef * vals + (1.0 - ef) * 0.5
      sat_v[slot][pl.ds(o, L)] = jnp.where(ev > 0.5, 1.0, 0.0).astype(
          jnp.float32)

  def start_scatter(slot):
    pltpu.async_copy(sat_v[slot], shared.at[fidx_v[slot]], sem_sc[slot],
                     add=True)

  def wait_scatter(slot):
    pltpu.make_async_copy(sat_v[slot], shared.at[fidx_v[slot]],
                          sem_sc[slot]).wait()

  start_in(0, 0)
  pltpu.async_copy(vp_hbm, vp_v, sem_vp)

  @pl.loop(0, SLICE // L)
  def _zero(i):
    zbuf[pl.ds(i * L, L)] = jnp.zeros((L,), jnp.float32)

  pltpu.sync_copy(zbuf, shared.at[pl.ds(s * SLICE, SLICE)])

  plsc.subcore_barrier()
  pltpu.make_async_copy(vp_hbm, vp_v, sem_vp).wait()

  wait_in(0)
  compute(0)
  start_in(1, 1)
  start_scatter(0)

  @pl.loop(1, NCHUNK - 1, step=2)
  def _pair(ci0):
    for b in range(2):
      ci = ci0 + b
      cur = (1 + b) % 2
      oth = 1 - cur
      wait_in(cur)
      compute(cur)
      wait_scatter(oth)
      start_in(ci + 1, oth)
      start_scatter(cur)

  wait_in(1)
  compute(1)
  wait_scatter(0)
  start_scatter(1)
  wait_scatter(1)

  plsc.subcore_barrier()
  pltpu.sync_copy(shared.at[pl.ds(s * SLICE, SLICE)], zbuf)
  pltpu.sync_copy(zbuf, out_hbm.at[pl.ds(c * F_PAD + s * SLICE, SLICE)])


def _tc_body(p_ref, bfm_ref, is_sat_ref, unsat_ref):
  nrow = F_PAD // 128
  total = p_ref[:nrow, :] + p_ref[nrow:, :]
  in_batch = bfm_ref[...] == 0
  clause_sat = jnp.logical_and(total > 0.0, in_batch)
  batch_values = jnp.sum(clause_sat.astype(jnp.int32))
  max_sat = jnp.sum(in_batch.astype(jnp.int32))
  is_sat_ref[...] = jnp.where(batch_values == max_sat, 1.0, 0.0).reshape(1, 1)
  unsat_ref[...] = (max_sat - batch_values).astype(jnp.float32).reshape(1, 1)


@jax.jit
def kernel(variable_prediction, label, graph_map, batch_variable_map,
           batch_function_map, edge_feature, meta_data):
  del label, batch_variable_map, meta_data
  vp = variable_prediction.reshape(V)
  ef = edge_feature.reshape(E)
  gm = graph_map.reshape(2 * E)

  assert NCHUNK % 2 == 0

  mesh = plsc.VectorSubcoreMesh(core_axis_name="c", subcore_axis_name="s")
  partials = pl.kernel(
      _sc_body,
      out_type=jax.ShapeDtypeStruct((NC * F_PAD,), jnp.float32),
      mesh=mesh,
      compiler_params=pltpu.CompilerParams(needs_layout_passes=False),
      scratch_types=[
          pltpu.VMEM((V,), jnp.float32),
          pltpu.VMEM((CH,), jnp.int32),
          pltpu.VMEM((CH,), jnp.int32),
          pltpu.VMEM((CH,), jnp.int32),
          pltpu.VMEM((CH,), jnp.int32),
          pltpu.VMEM((CH,), jnp.float32),
          pltpu.VMEM((CH,), jnp.float32),
          pltpu.VMEM((CH,), jnp.float32),
          pltpu.VMEM((CH,), jnp.float32),
          pltpu.VMEM((SLICE,), jnp.float32),
          pltpu.VMEM_SHARED((F_PAD,), jnp.float32),
          pltpu.SemaphoreType.DMA,
          pltpu.SemaphoreType.DMA,
          pltpu.SemaphoreType.DMA,
          pltpu.SemaphoreType.DMA,
          pltpu.SemaphoreType.DMA,
      ],
  )(gm, ef, vp)

  bfm = jnp.pad(batch_function_map, (0, F_PAD - F), constant_values=1)

  is_sat, unsat = pl.pallas_call(
      _tc_body,
      out_shape=(
          jax.ShapeDtypeStruct((1, 1), jnp.float32),
          jax.ShapeDtypeStruct((1, 1), jnp.float32),
      ),
  )(partials.reshape(2 * F_PAD // 128, 128),
    bfm.reshape(F_PAD // 128, 128))
  return is_sat, unsat


# speedup vs baseline: 1.2272x; 1.1462x over previous
import jax
import jax.numpy as jnp
from jax import lax
from jax.experimental import pallas as pl
from jax.experimental.pallas import tpu as pltpu
from jax.experimental.pallas import tpu_sc as plsc

NC = 2
NS = 16
NW = NC * NS
L = 16

V = 100000
F = 100000
E = 3200000

F_PAD = 100352
SLICE = F_PAD // NS

EPW = E // NW
CH = 4000
NCHUNK = EPW // CH


def _sc_body(gm_hbm, ef_hbm, vp_hbm, out_hbm,
             vp_v, vs0, vs1, fidx0, fidx1, ef0, ef1,
             shared, sem_in0, sem_in1, sem_sc0, sem_sc1, sem_vp):
  c = lax.axis_index("c")
  s = lax.axis_index("s")
  wid = c * NS + s
  base0 = wid * EPW
  vs_v = (vs0, vs1)
  fidx_v = (fidx0, fidx1)
  ef_v = (ef0, ef1)
  sem_in = (sem_in0, sem_in1)
  sem_sc = (sem_sc0, sem_sc1)

  def start_in(ci, slot):
    b = base0 + ci * CH
    b = jnp.where(b + CH <= E, b, 0)
    pltpu.async_copy(gm_hbm.at[pl.ds(b, CH)], vs_v[slot], sem_in[slot])
    pltpu.async_copy(gm_hbm.at[pl.ds(E + b, CH)], fidx_v[slot], sem_in[slot])
    pltpu.async_copy(ef_hbm.at[pl.ds(b, CH)], ef_v[slot], sem_in[slot])

  def wait_in(slot):
    pltpu.make_async_copy(gm_hbm.at[pl.ds(0, CH)], vs_v[slot],
                          sem_in[slot]).wait()
    pltpu.make_async_copy(gm_hbm.at[pl.ds(0, CH)], fidx_v[slot],
                          sem_in[slot]).wait()
    pltpu.make_async_copy(ef_hbm.at[pl.ds(0, CH)], ef_v[slot],
                          sem_in[slot]).wait()

  def compute(slot):
    @plsc.parallel_loop(0, CH // L, unroll=4)
    def _lanes(j):
      o = j * L
      vals = plsc.load_gather(vp_v, [vs_v[slot][pl.ds(o, L)]])
      ef = ef_v[slot][pl.ds(o, L)]
      ev = ef * vals + (1.0 - ef) * 0.5
      vs_v[slot][pl.ds(o, L)] = jnp.where(ev > 0.5, 1, 0).astype(jnp.int32)

  def start_scatter(slot):
    pltpu.async_copy(vs_v[slot], shared.at[fidx_v[slot]], sem_sc[slot],
                     add=True)

  def wait_scatter(slot):
    pltpu.make_async_copy(vs_v[slot], shared.at[fidx_v[slot]],
                          sem_sc[slot]).wait()

  start_in(0, 0)
  pltpu.async_copy(vp_hbm, vp_v, sem_vp)

  @pl.loop(0, CH // L)
  def _zero(i):
    vs1[pl.ds(i * L, L)] = jnp.zeros((L,), jnp.int32)

  pltpu.sync_copy(vs1, shared.at[pl.ds(s * SLICE, CH)])
  pltpu.sync_copy(vs1.at[pl.ds(0, SLICE - CH)],
                  shared.at[pl.ds(s * SLICE + CH, SLICE - CH)])

  plsc.subcore_barrier()
  pltpu.make_async_copy(vp_hbm, vp_v, sem_vp).wait()

  wait_in(0)
  compute(0)
  start_in(1, 1)
  start_scatter(0)

  @pl.loop(1, NCHUNK, step=2)
  def _pair(ci0):
    for b in range(2):
      ci = ci0 + b
      cur = (1 + b) % 2
      oth = 1 - cur
      wait_in(cur)
      compute(cur)
      wait_scatter(oth)
      start_in(ci + 1, oth)
      start_scatter(cur)

  wait_scatter(0)
  wait_in(1)

  plsc.subcore_barrier()
  pltpu.sync_copy(shared.at[pl.ds(s * SLICE, CH)], vs0)
  pltpu.sync_copy(shared.at[pl.ds(s * SLICE + CH, SLICE - CH)],
                  vs1.at[pl.ds(0, SLICE - CH)])
  pltpu.sync_copy(vs0, out_hbm.at[pl.ds(c * F_PAD + s * SLICE, CH)])
  pltpu.sync_copy(vs1.at[pl.ds(0, SLICE - CH)],
                  out_hbm.at[pl.ds(c * F_PAD + s * SLICE + CH, SLICE - CH)])


def _tc_body(p_ref, bfm_ref, is_sat_ref, unsat_ref):
  nrow = F_PAD // 128
  total = p_ref[:nrow, :] + p_ref[nrow:, :]
  in_batch = bfm_ref[...] == 0
  clause_sat = jnp.logical_and(total > 0, in_batch)
  batch_values = jnp.sum(clause_sat.astype(jnp.int32))
  max_sat = jnp.sum(in_batch.astype(jnp.int32))
  is_sat_ref[...] = jnp.where(batch_values == max_sat, 1.0, 0.0).reshape(1, 1)
  unsat_ref[...] = (max_sat - batch_values).astype(jnp.float32).reshape(1, 1)


@jax.jit
def kernel(variable_prediction, label, graph_map, batch_variable_map,
           batch_function_map, edge_feature, meta_data):
  del label, batch_variable_map, meta_data
  vp = variable_prediction.reshape(V)
  ef = edge_feature.reshape(E)
  gm = graph_map.reshape(2 * E)

  assert NCHUNK % 2 == 1

  mesh = plsc.VectorSubcoreMesh(core_axis_name="c", subcore_axis_name="s")
  partials = pl.kernel(
      _sc_body,
      out_type=jax.ShapeDtypeStruct((NC * F_PAD,), jnp.int32),
      mesh=mesh,
      compiler_params=pltpu.CompilerParams(needs_layout_passes=False),
      scratch_types=[
          pltpu.VMEM((V,), jnp.float32),
          pltpu.VMEM((CH,), jnp.int32),
          pltpu.VMEM((CH,), jnp.int32),
          pltpu.VMEM((CH,), jnp.int32),
          pltpu.VMEM((CH,), jnp.int32),
          pltpu.VMEM((CH,), jnp.float32),
          pltpu.VMEM((CH,), jnp.float32),
          pltpu.VMEM_SHARED((F_PAD,), jnp.int32),
          pltpu.SemaphoreType.DMA,
          pltpu.SemaphoreType.DMA,
          pltpu.SemaphoreType.DMA,
          pltpu.SemaphoreType.DMA,
          pltpu.SemaphoreType.DMA,
      ],
  )(gm, ef, vp)

  bfm = jnp.pad(batch_function_map, (0, F_PAD - F), constant_values=1)

  is_sat, unsat = pl.pallas_call(
      _tc_body,
      out_shape=(
          jax.ShapeDtypeStruct((1, 1), jnp.float32),
          jax.ShapeDtypeStruct((1, 1), jnp.float32),
      ),
  )(partials.reshape(2 * F_PAD // 128, 128),
    bfm.reshape(F_PAD // 128, 128))
  return is_sat, unsat
